# baseline (speedup 1.0000x reference)
import numpy as np
import jax
import jax.numpy as jnp
from jax.experimental import pallas as pl
from jax.experimental.pallas import tpu as pltpu

EPS = 1e-5
S = 2048
NV = 1024
D = 1024
NH = 16
DH = 64
NE = 8
DFF = 16
NI = 64
TB = 256
NTB = NV // TB
HPS = 4

_EXPAND = np.kron(np.eye(NE, dtype=np.float32), np.ones((1, DFF), np.float32))

_TN = (((0,), (0,)), ((), ()))
_NT = (((1,), (1,)), ((), ()))
_NN = (((1,), (0,)), ((), ()))
_WX = (((0,), (1,)), ((), ()))


def _bf(a):
    return a.astype(jnp.bfloat16)


def _f32dot(lhs, rhs, dims):
    return jax.lax.dot_general(lhs, rhs, dims,
                               preferred_element_type=jnp.float32)


def _ln(x, w, b):
    m = jnp.mean(x, axis=-1, keepdims=True)
    d = x - m
    v = jnp.mean(d * d, axis=-1, keepdims=True)
    return d * jax.lax.rsqrt(v + EPS) * w + b


def _emb_qkv_kernel(x_ref, rw_ref, rb_ref, ew_ref, eb_ref,
                    wq_ref, wk_ref, wv_ref, bq_ref, bk_ref, bv_ref,
                    x0_ref, q_ref, k_ref, v_ref, m_ref, s_ref,
                    ew_bf, wq_bf, wk_bf, wv_bf):
    j = pl.program_id(0)

    @pl.when(j == 0)
    def _():
        ew_bf[...] = _bf(ew_ref[...])
        wq_bf[...] = _bf(wq_ref[...])
        wk_bf[...] = _bf(wk_ref[...])
        wv_bf[...] = _bf(wv_ref[...])

    x = x_ref[...]
    m = jnp.mean(x, axis=0, keepdims=True)
    d = x - m
    var = jnp.mean(d * d, axis=0, keepdims=True)
    std = jnp.sqrt(var + EPS)
    xn = d / std * rw_ref[...] + rb_ref[...]
    xe = _f32dot(_bf(xn), ew_bf[...], _TN) + eb_ref[...]
    x0_ref[...] = xe
    xb = _bf(xe)
    q_ref[...] = _bf(_f32dot(wq_bf[...], xb, _WX) + bq_ref[...])
    k_ref[...] = _bf(_f32dot(wk_bf[...], xb, _WX) + bk_ref[...])
    v_ref[...] = _bf(_f32dot(wv_bf[...], xb, _WX) + bv_ref[...])
    m_ref[...] = m
    s_ref[...] = std


def _attn_kernel(q_ref, k_ref, v_ref, o_ref):
    ones = jnp.ones((1, NV), jnp.bfloat16)
    for i in range(HPS):
        sl = pl.ds(i * DH, DH)
        qt = q_ref[sl, :]
        kt = k_ref[sl, :]
        vt = v_ref[sl, :]
        s = _f32dot(qt, kt, _TN) * 0.125
        p = _bf(jnp.exp(s))
        sums = _f32dot(ones, p, _NT)
        ot = _f32dot(vt, p, _NT)
        o_ref[sl, :] = _bf(ot * (1.0 / sums))


def _beta_kernel(ins_ref, rm_ref, g2_ref, b_ref, beta_ref, rgam_ref, gam_ref):
    im = jnp.mean(ins_ref[...], axis=0, keepdims=True)
    beta_ref[0] = jnp.dot(im, b_ref[0], preferred_element_type=jnp.float32)

    @pl.when(pl.program_id(0) == 0)
    def _():
        rgam_ref[...] = jnp.dot(im, rm_ref[...],
                                preferred_element_type=jnp.float32)
        gam_ref[...] = _f32dot(im, g2_ref[...], _NT)


def _moe_body(x_blk, o_blk, wo_bf, bo, l1w, l1b, gw, rgam, gam, beta,
              gu_bf, dcat_bf, ex, l2w, l2b):
    att = _f32dot(o_blk, wo_bf, _TN)
    t = _ln(x_blk + bo + att, l1w, l1b)
    logits = jnp.dot(t, gw, preferred_element_type=jnp.float32) + rgam
    mx = jnp.max(logits, axis=-1, keepdims=True)
    e = jnp.exp(logits - mx)
    rw = e / jnp.sum(e, axis=-1, keepdims=True)
    m1 = jnp.max(rw, axis=-1, keepdims=True)
    is1 = rw >= m1
    m2 = jnp.max(jnp.where(is1, -1.0, rw), axis=-1, keepdims=True)
    cnt = jnp.sum(is1.astype(jnp.float32), axis=-1, keepdims=True)
    thr = jnp.where(cnt > 1.5, m1, m2)
    w = jnp.where(rw >= thr, rw, 0.0)
    scale = jnp.dot(w * gam, ex, preferred_element_type=jnp.float32)
    hgu = _f32dot(_bf(t), gu_bf, _NN)
    hg = hgu[:, :NE * DFF]
    hu = hgu[:, NE * DFF:]
    hact = hg * jax.nn.sigmoid(hg) * hu * scale
    moe = (_f32dot(_bf(hact), dcat_bf, _NN)
           + jnp.dot(w, beta, preferred_element_type=jnp.float32))
    return _ln(t + moe, l2w, l2b)


def _moe_qkv_kernel(x_ref, o_ref, wo_ref, bo_ref, l1w_ref, l1b_ref,
                    gw_ref, rgam_ref, gam_ref, beta_ref, gu_ref, dcat_ref,
                    ex_ref, l2w_ref, l2b_ref,
                    wq_ref, wk_ref, wv_ref, bq_ref, bk_ref, bv_ref,
                    x1_ref, q_ref, k_ref, v_ref,
                    wo_bf, gu_bf, dcat_bf, wq_bf, wk_bf, wv_bf):
    j = pl.program_id(0)

    @pl.when(j == 0)
    def _():
        wo_bf[...] = _bf(wo_ref[...])
        gu_bf[...] = _bf(gu_ref[...])
        dcat_bf[...] = _bf(dcat_ref[...])
        wq_bf[...] = _bf(wq_ref[...])
        wk_bf[...] = _bf(wk_ref[...])
        wv_bf[...] = _bf(wv_ref[...])

    x1 = _moe_body(x_ref[...], o_ref[...], wo_bf[...], bo_ref[...],
                   l1w_ref[...], l1b_ref[...], gw_ref[...], rgam_ref[...],
                   gam_ref[...], beta_ref[...], gu_bf[...], dcat_bf[...],
                   ex_ref[...], l2w_ref[...], l2b_ref[...])
    x1_ref[...] = x1
    xb = _bf(x1)
    q_ref[...] = _bf(_f32dot(wq_bf[...], xb, _WX) + bq_ref[...])
    k_ref[...] = _bf(_f32dot(wk_bf[...], xb, _WX) + bk_ref[...])
    v_ref[...] = _bf(_f32dot(wv_bf[...], xb, _WX) + bv_ref[...])


def _moe_final_kernel(x_ref, o_ref, wo_ref, bo_ref, l1w_ref, l1b_ref,
                      gw_ref, rgam_ref, gam_ref, beta_ref, gu_ref, dcat_ref,
                      ex_ref, l2w_ref, l2b_ref,
                      nw_ref, nb_ref, pw_ref, pb_ref, rw_ref, rb_ref,
                      m_ref, s_ref, out_ref,
                      wo_bf, gu_bf, dcat_bf, pw_bf):
    j = pl.program_id(0)

    @pl.when(j == 0)
    def _():
        wo_bf[...] = _bf(wo_ref[...])
        gu_bf[...] = _bf(gu_ref[...])
        dcat_bf[...] = _bf(dcat_ref[...])
        pw_bf[...] = _bf(pw_ref[...])

    x2 = _moe_body(x_ref[...], o_ref[...], wo_bf[...], bo_ref[...],
                   l1w_ref[...], l1b_ref[...], gw_ref[...], rgam_ref[...],
                   gam_ref[...], beta_ref[...], gu_bf[...], dcat_bf[...],
                   ex_ref[...], l2w_ref[...], l2b_ref[...])
    xl = _ln(x2, nw_ref[...], nb_ref[...])
    y = _f32dot(pw_bf[...], _bf(xl), _WX) + pb_ref[...]
    out_ref[...] = ((y - rb_ref[...]) / (rw_ref[...] + EPS * EPS)
                    * s_ref[...] + m_ref[...])


def _attention(q, k, v):
    return pl.pallas_call(
        _attn_kernel,
        grid=(NH // HPS,),
        in_specs=[
            pl.BlockSpec((HPS * DH, NV), lambda h: (h, 0)),
            pl.BlockSpec((HPS * DH, NV), lambda h: (h, 0)),
            pl.BlockSpec((HPS * DH, NV), lambda h: (h, 0)),
        ],
        out_specs=pl.BlockSpec((HPS * DH, NV), lambda h: (h, 0)),
        out_shape=jax.ShapeDtypeStruct((D, NV), jnp.bfloat16),
    )(q, k, v)


def kernel(x_enc, Ins_tk, params):
    p = params
    x2 = x_enc.reshape(S, NV)
    ins = Ins_tk.reshape(NI, D)
    row = lambda a: a.reshape(1, -1)
    col = lambda a: a.reshape(-1, 1)
    f32 = jnp.float32
    bf16 = jnp.bfloat16
    l1, l2 = p['layers']

    def res(shape):
        return pl.BlockSpec(shape, lambda j: tuple(0 for _ in shape))

    tok_rows = pl.BlockSpec((TB, D), lambda j: (j, 0))
    tok_cols = pl.BlockSpec((D, TB), lambda j: (0, j))
    tok_row1 = pl.BlockSpec((1, TB), lambda j: (0, j))

    x0, q1, k1, v1, mean, std = pl.pallas_call(
        _emb_qkv_kernel,
        grid=(NTB,),
        in_specs=[
            pl.BlockSpec((S, TB), lambda j: (0, j)),
            tok_row1, tok_row1,
            res((S, D)), res((1, D)),
            res((D, D)), res((D, D)), res((D, D)),
            res((D, 1)), res((D, 1)), res((D, 1)),
        ],
        out_specs=[tok_rows, tok_cols, tok_cols, tok_cols,
                   tok_row1, tok_row1],
        out_shape=[jax.ShapeDtypeStruct((NV, D), f32)]
        + [jax.ShapeDtypeStruct((D, NV), bf16)] * 3
        + [jax.ShapeDtypeStruct((1, NV), f32)] * 2,
        scratch_shapes=[pltpu.VMEM((S, D), bf16)]
        + [pltpu.VMEM((D, D), bf16)] * 3,
    )(x2, row(p['revin_w']), row(p['revin_b']), p['emb_W'], row(p['emb_b']),
      l1['Wq'], l1['Wk'], l1['Wv'],
      col(l1['bq']), col(l1['bk']), col(l1['bv']))

    def beta_call(lp):
        return pl.pallas_call(
            _beta_kernel,
            grid=(NE,),
            in_specs=[
                pl.BlockSpec((NI, D), lambda e: (0, 0)),
                pl.BlockSpec((D, NE), lambda e: (0, 0)),
                pl.BlockSpec((NE, D), lambda e: (0, 0)),
                pl.BlockSpec((1, D, D), lambda e: (e, 0, 0)),
            ],
            out_specs=[pl.BlockSpec((1, 1, D), lambda e: (e, 0, 0)),
                       pl.BlockSpec((1, NE), lambda e: (0, 0)),
                       pl.BlockSpec((1, NE), lambda e: (0, 0))],
            out_shape=[jax.ShapeDtypeStruct((NE, 1, D), f32),
                       jax.ShapeDtypeStruct((1, NE), f32),
                       jax.ShapeDtypeStruct((1, NE), f32)],
        )(ins, lp['rm_W'], lp['eilm_g'][..., 0], lp['eilm_b'])

    def moe_weights(lp):
        gu = jnp.concatenate([
            jnp.transpose(lp['exp_gate'], (1, 0, 2)).reshape(D, NE * DFF),
            jnp.transpose(lp['exp_up'], (1, 0, 2)).reshape(D, NE * DFF)],
            axis=1)
        dcat = lp['exp_down'].reshape(NE * DFF, D)
        return gu, dcat

    beta1, rgam1, gam1 = beta_call(l1)
    beta2, rgam2, gam2 = beta_call(l2)
    gu1, dcat1 = moe_weights(l1)
    gu2, dcat2 = moe_weights(l2)
    expand = jnp.asarray(_EXPAND)

    o1 = _attention(q1, k1, v1)

    moe_in_specs = [
        tok_rows,
        tok_cols,
        res((D, D)), res((1, D)), res((1, D)), res((1, D)),
        res((D, NE)), res((1, NE)), res((1, NE)), res((NE, D)),
        res((D, 2 * NE * DFF)), res((NE * DFF, D)), res((NE, NE * DFF)),
        res((1, D)), res((1, D)),
    ]
    moe_scratch = [pltpu.VMEM((D, D), bf16),
                   pltpu.VMEM((D, 2 * NE * DFF), bf16),
                   pltpu.VMEM((NE * DFF, D), bf16)]

    x1, q2, k2, v2 = pl.pallas_call(
        _moe_qkv_kernel,
        grid=(NTB,),
        in_specs=moe_in_specs + [
            res((D, D)), res((D, D)), res((D, D)),
            res((D, 1)), res((D, 1)), res((D, 1)),
        ],
        out_specs=[tok_rows, tok_cols, tok_cols, tok_cols],
        out_shape=[jax.ShapeDtypeStruct((NV, D), f32)]
        + [jax.ShapeDtypeStruct((D, NV), bf16)] * 3,
        scratch_shapes=moe_scratch + [pltpu.VMEM((D, D), bf16)] * 3,
    )(x0, o1, l1['Wo'], row(l1['bo']), row(l1['ln1_w']), row(l1['ln1_b']),
      l1['gate_W'], rgam1, gam1, beta1.reshape(NE, D), gu1, dcat1, expand,
      row(l1['ln2_w']), row(l1['ln2_b']),
      l2['Wq'], l2['Wk'], l2['Wv'],
      col(l2['bq']), col(l2['bk']), col(l2['bv']))

    o2 = _attention(q2, k2, v2)

    out = pl.pallas_call(
        _moe_final_kernel,
        grid=(NTB,),
        in_specs=moe_in_specs + [
            res((1, D)), res((1, D)),
            res((D, D)), res((D, 1)),
            tok_row1, tok_row1,
            tok_row1, tok_row1,
        ],
        out_specs=tok_cols,
        out_shape=jax.ShapeDtypeStruct((D, NV), f32),
        scratch_shapes=moe_scratch + [pltpu.VMEM((D, D), bf16)],
    )(x1, o2, l2['Wo'], row(l2['bo']), row(l2['ln1_w']), row(l2['ln1_b']),
      l2['gate_W'], rgam2, gam2, beta2.reshape(NE, D), gu2, dcat2, expand,
      row(l2['ln2_w']), row(l2['ln2_b']),
      row(p['enc_norm_w']), row(p['enc_norm_b']), p['proj_W'],
      col(p['proj_b']), row(p['revin_w']), row(p['revin_b']), mean, std)

    return out.reshape(1, D, NV)


# speedup vs baseline: 1.0248x; 1.0248x over previous
import numpy as np
import jax
import jax.numpy as jnp
from jax.experimental import pallas as pl
from jax.experimental.pallas import tpu as pltpu

EPS = 1e-5
S = 2048
NV = 1024
D = 1024
NH = 16
DH = 64
NE = 8
DFF = 16
NI = 64
TB = 512
NTB = NV // TB
HPS = 4

_EXPAND = np.kron(np.eye(NE, dtype=np.float32), np.ones((1, DFF), np.float32))

_TN = (((0,), (0,)), ((), ()))
_NT = (((1,), (1,)), ((), ()))
_NN = (((1,), (0,)), ((), ()))
_WX = (((0,), (1,)), ((), ()))


def _bf(a):
    return a.astype(jnp.bfloat16)


def _f32dot(lhs, rhs, dims):
    return jax.lax.dot_general(lhs, rhs, dims,
                               preferred_element_type=jnp.float32)


def _ln(x, w, b):
    m = jnp.mean(x, axis=-1, keepdims=True)
    d = x - m
    v = jnp.mean(d * d, axis=-1, keepdims=True)
    return d * jax.lax.rsqrt(v + EPS) * w + b


def _emb_qkv_kernel(x_ref, rw_ref, rb_ref, ew_ref, eb_ref,
                    wq_ref, wk_ref, wv_ref, bq_ref, bk_ref, bv_ref,
                    x0_ref, q_ref, k_ref, v_ref, m_ref, s_ref,
                    ew_bf, wq_bf, wk_bf, wv_bf):
    j = pl.program_id(0)

    @pl.when(j == 0)
    def _():
        ew_bf[...] = _bf(ew_ref[...])
        wq_bf[...] = _bf(wq_ref[...])
        wk_bf[...] = _bf(wk_ref[...])
        wv_bf[...] = _bf(wv_ref[...])

    x = x_ref[...]
    m = jnp.mean(x, axis=0, keepdims=True)
    d = x - m
    var = jnp.mean(d * d, axis=0, keepdims=True)
    std = jnp.sqrt(var + EPS)
    xn = d / std * rw_ref[...] + rb_ref[...]
    xe = _f32dot(_bf(xn), ew_bf[...], _TN) + eb_ref[...]
    x0_ref[...] = xe
    xb = _bf(xe)
    q_ref[...] = _bf((_f32dot(wq_bf[...], xb, _WX) + bq_ref[...]) * 0.125)
    k_ref[...] = _bf(_f32dot(wk_bf[...], xb, _WX) + bk_ref[...])
    v_ref[...] = _bf(_f32dot(wv_bf[...], xb, _WX) + bv_ref[...])
    m_ref[...] = m
    s_ref[...] = std


def _attn_kernel(q_ref, k_ref, v_ref, o_ref):
    ones = jnp.ones((1, NV), jnp.bfloat16)
    for i in range(HPS):
        sl = pl.ds(i * DH, DH)
        qt = q_ref[sl, :]
        kt = k_ref[sl, :]
        vt = v_ref[sl, :]
        s = _f32dot(qt, kt, _TN)
        p = jnp.exp(_bf(s))
        sums = _f32dot(ones, p, _NT)
        ot = _f32dot(vt, p, _NT)
        o_ref[sl, :] = _bf(ot * (1.0 / sums))


def _beta_kernel(ins_ref, rm_ref, g2_ref, b_ref, beta_ref, rgam_ref, gam_ref):
    im = jnp.mean(ins_ref[...], axis=0, keepdims=True)
    beta_ref[0] = jnp.dot(im, b_ref[0], preferred_element_type=jnp.float32)

    @pl.when(pl.program_id(0) == 0)
    def _():
        rgam_ref[...] = jnp.dot(im, rm_ref[...],
                                preferred_element_type=jnp.float32)
        gam_ref[...] = _f32dot(im, g2_ref[...], _NT)


def _moe_body(x_blk, o_blk, wo_bf, bo, l1w, l1b, gw, rgam, gam, beta,
              gu_bf, dcat_bf, ex, l2w, l2b):
    att = _f32dot(o_blk, wo_bf, _TN)
    t = _ln(x_blk + bo + att, l1w, l1b)
    logits = jnp.dot(t, gw, preferred_element_type=jnp.float32) + rgam
    mx = jnp.max(logits, axis=-1, keepdims=True)
    e = jnp.exp(logits - mx)
    rw = e / jnp.sum(e, axis=-1, keepdims=True)
    m1 = jnp.max(rw, axis=-1, keepdims=True)
    is1 = rw >= m1
    m2 = jnp.max(jnp.where(is1, -1.0, rw), axis=-1, keepdims=True)
    cnt = jnp.sum(is1.astype(jnp.float32), axis=-1, keepdims=True)
    thr = jnp.where(cnt > 1.5, m1, m2)
    w = jnp.where(rw >= thr, rw, 0.0)
    scale = jnp.dot(w * gam, ex, preferred_element_type=jnp.float32)
    hgu = _f32dot(_bf(t), gu_bf, _NN)
    hg = hgu[:, :NE * DFF]
    hu = hgu[:, NE * DFF:]
    hact = hg * jax.nn.sigmoid(hg) * hu * scale
    moe = (_f32dot(_bf(hact), dcat_bf, _NN)
           + jnp.dot(w, beta, preferred_element_type=jnp.float32))
    return _ln(t + moe, l2w, l2b)


def _moe_qkv_kernel(x_ref, o_ref, wo_ref, bo_ref, l1w_ref, l1b_ref,
                    gw_ref, rgam_ref, gam_ref, beta_ref, gu_ref, dcat_ref,
                    ex_ref, l2w_ref, l2b_ref,
                    wq_ref, wk_ref, wv_ref, bq_ref, bk_ref, bv_ref,
                    x1_ref, q_ref, k_ref, v_ref,
                    wo_bf, gu_bf, dcat_bf, wq_bf, wk_bf, wv_bf):
    j = pl.program_id(0)

    @pl.when(j == 0)
    def _():
        wo_bf[...] = _bf(wo_ref[...])
        gu_bf[...] = _bf(gu_ref[...])
        dcat_bf[...] = _bf(dcat_ref[...])
        wq_bf[...] = _bf(wq_ref[...])
        wk_bf[...] = _bf(wk_ref[...])
        wv_bf[...] = _bf(wv_ref[...])

    x1 = _moe_body(x_ref[...], o_ref[...], wo_bf[...], bo_ref[...],
                   l1w_ref[...], l1b_ref[...], gw_ref[...], rgam_ref[...],
                   gam_ref[...], beta_ref[...], gu_bf[...], dcat_bf[...],
                   ex_ref[...], l2w_ref[...], l2b_ref[...])
    x1_ref[...] = x1
    xb = _bf(x1)
    q_ref[...] = _bf((_f32dot(wq_bf[...], xb, _WX) + bq_ref[...]) * 0.125)
    k_ref[...] = _bf(_f32dot(wk_bf[...], xb, _WX) + bk_ref[...])
    v_ref[...] = _bf(_f32dot(wv_bf[...], xb, _WX) + bv_ref[...])


def _moe_final_kernel(x_ref, o_ref, wo_ref, bo_ref, l1w_ref, l1b_ref,
                      gw_ref, rgam_ref, gam_ref, beta_ref, gu_ref, dcat_ref,
                      ex_ref, l2w_ref, l2b_ref,
                      nw_ref, nb_ref, pw_ref, pb_ref, rw_ref, rb_ref,
                      m_ref, s_ref, out_ref,
                      wo_bf, gu_bf, dcat_bf, pw_bf):
    j = pl.program_id(0)

    @pl.when(j == 0)
    def _():
        wo_bf[...] = _bf(wo_ref[...])
        gu_bf[...] = _bf(gu_ref[...])
        dcat_bf[...] = _bf(dcat_ref[...])
        pw_bf[...] = _bf(pw_ref[...])

    x2 = _moe_body(x_ref[...], o_ref[...], wo_bf[...], bo_ref[...],
                   l1w_ref[...], l1b_ref[...], gw_ref[...], rgam_ref[...],
                   gam_ref[...], beta_ref[...], gu_bf[...], dcat_bf[...],
                   ex_ref[...], l2w_ref[...], l2b_ref[...])
    xl = _ln(x2, nw_ref[...], nb_ref[...])
    y = _f32dot(pw_bf[...], _bf(xl), _WX) + pb_ref[...]
    out_ref[...] = ((y - rb_ref[...]) / (rw_ref[...] + EPS * EPS)
                    * s_ref[...] + m_ref[...])


def _attention(q, k, v):
    return pl.pallas_call(
        _attn_kernel,
        grid=(NH // HPS,),
        in_specs=[
            pl.BlockSpec((HPS * DH, NV), lambda h: (h, 0)),
            pl.BlockSpec((HPS * DH, NV), lambda h: (h, 0)),
            pl.BlockSpec((HPS * DH, NV), lambda h: (h, 0)),
        ],
        out_specs=pl.BlockSpec((HPS * DH, NV), lambda h: (h, 0)),
        out_shape=jax.ShapeDtypeStruct((D, NV), jnp.bfloat16),
    )(q, k, v)


def kernel(x_enc, Ins_tk, params):
    p = params
    x2 = x_enc.reshape(S, NV)
    ins = Ins_tk.reshape(NI, D)
    row = lambda a: a.reshape(1, -1)
    col = lambda a: a.reshape(-1, 1)
    f32 = jnp.float32
    bf16 = jnp.bfloat16
    l1, l2 = p['layers']

    def res(shape):
        return pl.BlockSpec(shape, lambda j: tuple(0 for _ in shape))

    tok_rows = pl.BlockSpec((TB, D), lambda j: (j, 0))
    tok_cols = pl.BlockSpec((D, TB), lambda j: (0, j))
    tok_row1 = pl.BlockSpec((1, TB), lambda j: (0, j))

    x0, q1, k1, v1, mean, std = pl.pallas_call(
        _emb_qkv_kernel,
        grid=(NTB,),
        in_specs=[
            pl.BlockSpec((S, TB), lambda j: (0, j)),
            tok_row1, tok_row1,
            res((S, D)), res((1, D)),
            res((D, D)), res((D, D)), res((D, D)),
            res((D, 1)), res((D, 1)), res((D, 1)),
        ],
        out_specs=[tok_rows, tok_cols, tok_cols, tok_cols,
                   tok_row1, tok_row1],
        out_shape=[jax.ShapeDtypeStruct((NV, D), f32)]
        + [jax.ShapeDtypeStruct((D, NV), bf16)] * 3
        + [jax.ShapeDtypeStruct((1, NV), f32)] * 2,
        scratch_shapes=[pltpu.VMEM((S, D), bf16)]
        + [pltpu.VMEM((D, D), bf16)] * 3,
    )(x2, row(p['revin_w']), row(p['revin_b']), p['emb_W'], row(p['emb_b']),
      l1['Wq'], l1['Wk'], l1['Wv'],
      col(l1['bq']), col(l1['bk']), col(l1['bv']))

    def beta_call(lp):
        return pl.pallas_call(
            _beta_kernel,
            grid=(NE,),
            in_specs=[
                pl.BlockSpec((NI, D), lambda e: (0, 0)),
                pl.BlockSpec((D, NE), lambda e: (0, 0)),
                pl.BlockSpec((NE, D), lambda e: (0, 0)),
                pl.BlockSpec((1, D, D), lambda e: (e, 0, 0)),
            ],
            out_specs=[pl.BlockSpec((1, 1, D), lambda e: (e, 0, 0)),
                       pl.BlockSpec((1, NE), lambda e: (0, 0)),
                       pl.BlockSpec((1, NE), lambda e: (0, 0))],
            out_shape=[jax.ShapeDtypeStruct((NE, 1, D), f32),
                       jax.ShapeDtypeStruct((1, NE), f32),
                       jax.ShapeDtypeStruct((1, NE), f32)],
        )(ins, lp['rm_W'], lp['eilm_g'][..., 0], lp['eilm_b'])

    def moe_weights(lp):
        gu = jnp.concatenate([
            jnp.transpose(lp['exp_gate'], (1, 0, 2)).reshape(D, NE * DFF),
            jnp.transpose(lp['exp_up'], (1, 0, 2)).reshape(D, NE * DFF)],
            axis=1)
        dcat = lp['exp_down'].reshape(NE * DFF, D)
        return gu, dcat

    beta1, rgam1, gam1 = beta_call(l1)
    beta2, rgam2, gam2 = beta_call(l2)
    gu1, dcat1 = moe_weights(l1)
    gu2, dcat2 = moe_weights(l2)
    expand = jnp.asarray(_EXPAND)

    o1 = _attention(q1, k1, v1)

    moe_in_specs = [
        tok_rows,
        tok_cols,
        res((D, D)), res((1, D)), res((1, D)), res((1, D)),
        res((D, NE)), res((1, NE)), res((1, NE)), res((NE, D)),
        res((D, 2 * NE * DFF)), res((NE * DFF, D)), res((NE, NE * DFF)),
        res((1, D)), res((1, D)),
    ]
    moe_scratch = [pltpu.VMEM((D, D), bf16),
                   pltpu.VMEM((D, 2 * NE * DFF), bf16),
                   pltpu.VMEM((NE * DFF, D), bf16)]

    x1, q2, k2, v2 = pl.pallas_call(
        _moe_qkv_kernel,
        grid=(NTB,),
        in_specs=moe_in_specs + [
            res((D, D)), res((D, D)), res((D, D)),
            res((D, 1)), res((D, 1)), res((D, 1)),
        ],
        out_specs=[tok_rows, tok_cols, tok_cols, tok_cols],
        out_shape=[jax.ShapeDtypeStruct((NV, D), f32)]
        + [jax.ShapeDtypeStruct((D, NV), bf16)] * 3,
        scratch_shapes=moe_scratch + [pltpu.VMEM((D, D), bf16)] * 3,
    )(x0, o1, l1['Wo'], row(l1['bo']), row(l1['ln1_w']), row(l1['ln1_b']),
      l1['gate_W'], rgam1, gam1, beta1.reshape(NE, D), gu1, dcat1, expand,
      row(l1['ln2_w']), row(l1['ln2_b']),
      l2['Wq'], l2['Wk'], l2['Wv'],
      col(l2['bq']), col(l2['bk']), col(l2['bv']))

    o2 = _attention(q2, k2, v2)

    out = pl.pallas_call(
        _moe_final_kernel,
        grid=(NTB,),
        in_specs=moe_in_specs + [
            res((1, D)), res((1, D)),
            res((D, D)), res((D, 1)),
            tok_row1, tok_row1,
            tok_row1, tok_row1,
        ],
        out_specs=tok_cols,
        out_shape=jax.ShapeDtypeStruct((D, NV), f32),
        scratch_shapes=moe_scratch + [pltpu.VMEM((D, D), bf16)],
    )(x1, o2, l2['Wo'], row(l2['bo']), row(l2['ln1_w']), row(l2['ln1_b']),
      l2['gate_W'], rgam2, gam2, beta2.reshape(NE, D), gu2, dcat2, expand,
      row(l2['ln2_w']), row(l2['ln2_b']),
      row(p['enc_norm_w']), row(p['enc_norm_b']), p['proj_W'],
      col(p['proj_b']), row(p['revin_w']), row(p['revin_b']), mean, std)

    return out.reshape(1, D, NV)


# speedup vs baseline: 1.0499x; 1.0246x over previous
import functools
import numpy as np
import jax
import jax.numpy as jnp
from jax.experimental import pallas as pl
from jax.experimental.pallas import tpu as pltpu
from jax.experimental.pallas import tpu_sc as plsc

EPS = 1e-5
S = 2048
NV = 1024
D = 1024
NH = 16
DH = 64
NE = 8
DFF = 16
NI = 64
TB = 512
NTB = NV // TB
HPS = 4

_EXPAND = np.kron(np.eye(NE, dtype=np.float32), np.ones((1, DFF), np.float32))

_TN = (((0,), (0,)), ((), ()))
_NT = (((1,), (1,)), ((), ()))
_NN = (((1,), (0,)), ((), ()))
_WX = (((0,), (1,)), ((), ()))


def _bf(a):
    return a.astype(jnp.bfloat16)


def _f32dot(lhs, rhs, dims):
    return jax.lax.dot_general(lhs, rhs, dims,
                               preferred_element_type=jnp.float32)


def _ln(x, w, b):
    m = jnp.mean(x, axis=-1, keepdims=True)
    d = x - m
    v = jnp.mean(d * d, axis=-1, keepdims=True)
    return d * jax.lax.rsqrt(v + EPS) * w + b


def _emb_qkv_kernel(x_ref, rw_ref, rb_ref, ew_ref, eb_ref,
                    wq_ref, wk_ref, wv_ref, bq_ref, bk_ref, bv_ref,
                    x0_ref, q_ref, k_ref, v_ref, m_ref, s_ref,
                    ew_bf, wq_bf, wk_bf, wv_bf):
    j = pl.program_id(0)

    @pl.when(j == 0)
    def _():
        ew_bf[...] = _bf(ew_ref[...])
        wq_bf[...] = _bf(wq_ref[...])
        wk_bf[...] = _bf(wk_ref[...])
        wv_bf[...] = _bf(wv_ref[...])

    x = x_ref[...]
    m = jnp.mean(x, axis=0, keepdims=True)
    d = x - m
    var = jnp.mean(d * d, axis=0, keepdims=True)
    std = jnp.sqrt(var + EPS)
    xn = d / std * rw_ref[...] + rb_ref[...]
    xe = _f32dot(_bf(xn), ew_bf[...], _TN) + eb_ref[...]
    x0_ref[...] = xe
    xb = _bf(xe)
    q_ref[...] = _bf((_f32dot(wq_bf[...], xb, _WX) + bq_ref[...]) * 0.125)
    k_ref[...] = _bf(_f32dot(wk_bf[...], xb, _WX) + bk_ref[...])
    v_ref[...] = _bf(_f32dot(wv_bf[...], xb, _WX) + bv_ref[...])
    m_ref[...] = m
    s_ref[...] = std


def _attn_kernel(q_ref, k_ref, v_ref, o_ref):
    ones = jnp.ones((1, NV), jnp.bfloat16)
    for i in range(HPS):
        sl = pl.ds(i * DH, DH)
        qt = q_ref[sl, :]
        kt = k_ref[sl, :]
        vt = v_ref[sl, :]
        s = _f32dot(qt, kt, _TN)
        p = jnp.exp(_bf(s))
        sums = _f32dot(ones, p, _NT)
        ot = _f32dot(vt, p, _NT)
        o_ref[sl, :] = _bf(ot * (1.0 / sums))


def _ins_kernel(ins_ref, rm_ref, g2_ref, im_ref, rgam_ref, gam_ref):
    im = jnp.mean(ins_ref[...], axis=0, keepdims=True)
    im_ref[...] = im
    rgam_ref[...] = jnp.dot(im, rm_ref[...], preferred_element_type=jnp.float32)
    gam_ref[...] = _f32dot(im, g2_ref[...], _NT)


_KC = 128
_DQ = 256
_NLG = _DQ // 16


def _sc_beta(im, eilm_b):
    mesh = plsc.VectorSubcoreMesh(core_axis_name="c", subcore_axis_name="s")

    @functools.partial(
        pl.kernel, mesh=mesh,
        out_type=jax.ShapeDtypeStruct((NE, D), jnp.float32),
        compiler_params=pltpu.CompilerParams(needs_layout_passes=False),
        scratch_types=[
            pltpu.VMEM((D,), jnp.float32),
            pltpu.VMEM((_KC, _DQ), jnp.float32),
            pltpu.VMEM((_DQ,), jnp.float32),
            pltpu.SemaphoreType.DMA,
        ],
    )
    def sc_kernel(im_hbm, b_hbm, beta_hbm, im_v, buf, acc_v, sem):
        wid = jax.lax.axis_index("s") * 2 + jax.lax.axis_index("c")
        e = wid // 4
        q = wid % 4
        pltpu.sync_copy(im_hbm.at[0], im_v)
        zero = jnp.zeros((16,), jnp.float32)

        def chunk_body(c, accs):
            pltpu.async_copy(
                b_hbm.at[e, pl.ds(c * _KC, _KC), pl.ds(q * _DQ, _DQ)],
                buf, sem).wait()

            def kbody(kl, accs):
                idx = jnp.zeros((16,), jnp.int32) + (c * _KC + kl)
                bc = plsc.load_gather(im_v, [idx])
                return tuple(accs[l] + bc * buf[kl, pl.ds(l * 16, 16)]
                             for l in range(_NLG))

            return jax.lax.fori_loop(0, _KC, kbody, accs)

        accs = jax.lax.fori_loop(0, D // _KC, chunk_body, (zero,) * _NLG)
        for l in range(_NLG):
            acc_v[pl.ds(l * 16, 16)] = accs[l]
        pltpu.sync_copy(acc_v, beta_hbm.at[e, pl.ds(q * _DQ, _DQ)])

    return sc_kernel(im, eilm_b)


def _moe_body(x_blk, o_blk, wo_bf, bo, l1w, l1b, gw, rgam, gam, beta,
              gu_bf, dcat_bf, ex, l2w, l2b):
    att = _f32dot(o_blk, wo_bf, _TN)
    t = _ln(x_blk + bo + att, l1w, l1b)
    logits = jnp.dot(t, gw, preferred_element_type=jnp.float32) + rgam
    mx = jnp.max(logits, axis=-1, keepdims=True)
    e = jnp.exp(logits - mx)
    rw = e / jnp.sum(e, axis=-1, keepdims=True)
    m1 = jnp.max(rw, axis=-1, keepdims=True)
    is1 = rw >= m1
    m2 = jnp.max(jnp.where(is1, -1.0, rw), axis=-1, keepdims=True)
    cnt = jnp.sum(is1.astype(jnp.float32), axis=-1, keepdims=True)
    thr = jnp.where(cnt > 1.5, m1, m2)
    w = jnp.where(rw >= thr, rw, 0.0)
    scale = jnp.dot(w * gam, ex, preferred_element_type=jnp.float32)
    hgu = _f32dot(_bf(t), gu_bf, _NN)
    hg = hgu[:, :NE * DFF]
    hu = hgu[:, NE * DFF:]
    hact = hg * jax.nn.sigmoid(hg) * hu * scale
    moe = (_f32dot(_bf(hact), dcat_bf, _NN)
           + jnp.dot(w, beta, preferred_element_type=jnp.float32))
    return _ln(t + moe, l2w, l2b)


def _moe_qkv_kernel(x_ref, o_ref, wo_ref, bo_ref, l1w_ref, l1b_ref,
                    gw_ref, rgam_ref, gam_ref, beta_ref, gu_ref, dcat_ref,
                    ex_ref, l2w_ref, l2b_ref,
                    wq_ref, wk_ref, wv_ref, bq_ref, bk_ref, bv_ref,
                    x1_ref, q_ref, k_ref, v_ref,
                    wo_bf, gu_bf, dcat_bf, wq_bf, wk_bf, wv_bf):
    j = pl.program_id(0)

    @pl.when(j == 0)
    def _():
        wo_bf[...] = _bf(wo_ref[...])
        gu_bf[...] = _bf(gu_ref[...])
        dcat_bf[...] = _bf(dcat_ref[...])
        wq_bf[...] = _bf(wq_ref[...])
        wk_bf[...] = _bf(wk_ref[...])
        wv_bf[...] = _bf(wv_ref[...])

    x1 = _moe_body(x_ref[...], o_ref[...], wo_bf[...], bo_ref[...],
                   l1w_ref[...], l1b_ref[...], gw_ref[...], rgam_ref[...],
                   gam_ref[...], beta_ref[...], gu_bf[...], dcat_bf[...],
                   ex_ref[...], l2w_ref[...], l2b_ref[...])
    x1_ref[...] = x1
    xb = _bf(x1)
    q_ref[...] = _bf((_f32dot(wq_bf[...], xb, _WX) + bq_ref[...]) * 0.125)
    k_ref[...] = _bf(_f32dot(wk_bf[...], xb, _WX) + bk_ref[...])
    v_ref[...] = _bf(_f32dot(wv_bf[...], xb, _WX) + bv_ref[...])


def _moe_final_kernel(x_ref, o_ref, wo_ref, bo_ref, l1w_ref, l1b_ref,
                      gw_ref, rgam_ref, gam_ref, beta_ref, gu_ref, dcat_ref,
                      ex_ref, l2w_ref, l2b_ref,
                      nw_ref, nb_ref, pw_ref, pb_ref, rw_ref, rb_ref,
                      m_ref, s_ref, out_ref,
                      wo_bf, gu_bf, dcat_bf, pw_bf):
    j = pl.program_id(0)

    @pl.when(j == 0)
    def _():
        wo_bf[...] = _bf(wo_ref[...])
        gu_bf[...] = _bf(gu_ref[...])
        dcat_bf[...] = _bf(dcat_ref[...])
        pw_bf[...] = _bf(pw_ref[...])

    x2 = _moe_body(x_ref[...], o_ref[...], wo_bf[...], bo_ref[...],
                   l1w_ref[...], l1b_ref[...], gw_ref[...], rgam_ref[...],
                   gam_ref[...], beta_ref[...], gu_bf[...], dcat_bf[...],
                   ex_ref[...], l2w_ref[...], l2b_ref[...])
    xl = _ln(x2, nw_ref[...], nb_ref[...])
    y = _f32dot(pw_bf[...], _bf(xl), _WX) + pb_ref[...]
    out_ref[...] = ((y - rb_ref[...]) / (rw_ref[...] + EPS * EPS)
                    * s_ref[...] + m_ref[...])


def _attention(q, k, v):
    return pl.pallas_call(
        _attn_kernel,
        grid=(NH // HPS,),
        in_specs=[
            pl.BlockSpec((HPS * DH, NV), lambda h: (h, 0)),
            pl.BlockSpec((HPS * DH, NV), lambda h: (h, 0)),
            pl.BlockSpec((HPS * DH, NV), lambda h: (h, 0)),
        ],
        out_specs=pl.BlockSpec((HPS * DH, NV), lambda h: (h, 0)),
        out_shape=jax.ShapeDtypeStruct((D, NV), jnp.bfloat16),
    )(q, k, v)


def kernel(x_enc, Ins_tk, params):
    p = params
    x2 = x_enc.reshape(S, NV)
    ins = Ins_tk.reshape(NI, D)
    row = lambda a: a.reshape(1, -1)
    col = lambda a: a.reshape(-1, 1)
    f32 = jnp.float32
    bf16 = jnp.bfloat16
    l1, l2 = p['layers']

    def res(shape):
        return pl.BlockSpec(shape, lambda j: tuple(0 for _ in shape))

    tok_rows = pl.BlockSpec((TB, D), lambda j: (j, 0))
    tok_cols = pl.BlockSpec((D, TB), lambda j: (0, j))
    tok_row1 = pl.BlockSpec((1, TB), lambda j: (0, j))

    x0, q1, k1, v1, mean, std = pl.pallas_call(
        _emb_qkv_kernel,
        grid=(NTB,),
        in_specs=[
            pl.BlockSpec((S, TB), lambda j: (0, j)),
            tok_row1, tok_row1,
            res((S, D)), res((1, D)),
            res((D, D)), res((D, D)), res((D, D)),
            res((D, 1)), res((D, 1)), res((D, 1)),
        ],
        out_specs=[tok_rows, tok_cols, tok_cols, tok_cols,
                   tok_row1, tok_row1],
        out_shape=[jax.ShapeDtypeStruct((NV, D), f32)]
        + [jax.ShapeDtypeStruct((D, NV), bf16)] * 3
        + [jax.ShapeDtypeStruct((1, NV), f32)] * 2,
        scratch_shapes=[pltpu.VMEM((S, D), bf16)]
        + [pltpu.VMEM((D, D), bf16)] * 3,
    )(x2, row(p['revin_w']), row(p['revin_b']), p['emb_W'], row(p['emb_b']),
      l1['Wq'], l1['Wk'], l1['Wv'],
      col(l1['bq']), col(l1['bk']), col(l1['bv']))

    def beta_call(lp):
        im, rgam, gam = pl.pallas_call(
            _ins_kernel,
            out_shape=[jax.ShapeDtypeStruct((1, D), f32),
                       jax.ShapeDtypeStruct((1, NE), f32),
                       jax.ShapeDtypeStruct((1, NE), f32)],
        )(ins, lp['rm_W'], lp['eilm_g'][..., 0])
        beta = _sc_beta(im, lp['eilm_b'])
        return beta, rgam, gam

    def moe_weights(lp):
        gu = jnp.concatenate([
            jnp.transpose(lp['exp_gate'], (1, 0, 2)).reshape(D, NE * DFF),
            jnp.transpose(lp['exp_up'], (1, 0, 2)).reshape(D, NE * DFF)],
            axis=1)
        dcat = lp['exp_down'].reshape(NE * DFF, D)
        return gu, dcat

    beta1, rgam1, gam1 = beta_call(l1)
    beta2, rgam2, gam2 = beta_call(l2)
    gu1, dcat1 = moe_weights(l1)
    gu2, dcat2 = moe_weights(l2)
    expand = jnp.asarray(_EXPAND)

    o1 = _attention(q1, k1, v1)

    moe_in_specs = [
        tok_rows,
        tok_cols,
        res((D, D)), res((1, D)), res((1, D)), res((1, D)),
        res((D, NE)), res((1, NE)), res((1, NE)), res((NE, D)),
        res((D, 2 * NE * DFF)), res((NE * DFF, D)), res((NE, NE * DFF)),
        res((1, D)), res((1, D)),
    ]
    moe_scratch = [pltpu.VMEM((D, D), bf16),
                   pltpu.VMEM((D, 2 * NE * DFF), bf16),
                   pltpu.VMEM((NE * DFF, D), bf16)]

    x1, q2, k2, v2 = pl.pallas_call(
        _moe_qkv_kernel,
        grid=(NTB,),
        in_specs=moe_in_specs + [
            res((D, D)), res((D, D)), res((D, D)),
            res((D, 1)), res((D, 1)), res((D, 1)),
        ],
        out_specs=[tok_rows, tok_cols, tok_cols, tok_cols],
        out_shape=[jax.ShapeDtypeStruct((NV, D), f32)]
        + [jax.ShapeDtypeStruct((D, NV), bf16)] * 3,
        scratch_shapes=moe_scratch + [pltpu.VMEM((D, D), bf16)] * 3,
    )(x0, o1, l1['Wo'], row(l1['bo']), row(l1['ln1_w']), row(l1['ln1_b']),
      l1['gate_W'], rgam1, gam1, beta1, gu1, dcat1, expand,
      row(l1['ln2_w']), row(l1['ln2_b']),
      l2['Wq'], l2['Wk'], l2['Wv'],
      col(l2['bq']), col(l2['bk']), col(l2['bv']))

    o2 = _attention(q2, k2, v2)

    out = pl.pallas_call(
        _moe_final_kernel,
        grid=(NTB,),
        in_specs=moe_in_specs + [
            res((1, D)), res((1, D)),
            res((D, D)), res((D, 1)),
            tok_row1, tok_row1,
            tok_row1, tok_row1,
        ],
        out_specs=tok_cols,
        out_shape=jax.ShapeDtypeStruct((D, NV), f32),
        scratch_shapes=moe_scratch + [pltpu.VMEM((D, D), bf16)],
    )(x1, o2, l2['Wo'], row(l2['bo']), row(l2['ln1_w']), row(l2['ln1_b']),
      l2['gate_W'], rgam2, gam2, beta2, gu2, dcat2, expand,
      row(l2['ln2_w']), row(l2['ln2_b']),
      row(p['enc_norm_w']), row(p['enc_norm_b']), p['proj_W'],
      col(p['proj_b']), row(p['revin_w']), row(p['revin_b']), mean, std)

    return out.reshape(1, D, NV)
